# baseline (speedup 1.0000x reference)
import functools

import jax
import jax.numpy as jnp
from jax import lax
from jax.experimental import pallas as pl
from jax.experimental.pallas import tpu as pltpu
from jax.experimental.pallas import tpu_sc as plsc

N_USERS = 30000
N_ITEMS = 20000
N = N_USERS + N_ITEMS
E = 800000
D = 64
H = 32
NC = 2
NS = 16
K = 128
SB = 8
CT = SB * (-(-E // (NS * K * SB)))
NSUP = CT // SB
E_PAD = NS * K * CT
NP = 50048
NR = NP // NS
ZR = 136
NB = NR // ZR


def _sc_body(x0, idata, Y,
             acc, idxc, idxr, vbuf, gbuf0, gbuf1, sbuf0, sbuf1, zbuf,
             gsem0, gsem1, ssem0, ssem1, zsem, isem):
    y1 = Y.at[0]
    y2 = Y.at[1]
    y3 = Y.at[2]
    mn = Y.at[3]
    mie = Y.at[4]
    cid = lax.axis_index("c")
    sid = lax.axis_index("s")
    row0 = sid * NR
    base_col = cid * NP
    gbufs = (gbuf0, gbuf1)
    sbufs = (sbuf0, sbuf1)
    gsems = (gsem0, gsem1)
    ssems = (ssem0, ssem1)
    cbase = sid * CT

    zero16 = jnp.zeros((16,), jnp.float32)
    for r in range(ZR):
        for h in range(0, H, 16):
            zbuf[r, pl.ds(h, 16)] = zero16

    def fetch_super(s, p):
        c0 = (cbase + s * SB) * K
        pltpu.async_copy(idata.at[pl.ds(cid * E_PAD + c0, SB * K)],
                         idxc.at[p], isem)
        for j in range(SB):
            pltpu.async_copy(idata.at[pl.ds(2 * E_PAD + c0 + j * K, K)],
                             idxr.at[p, j], isem)
        pltpu.async_copy(idata.at[pl.ds(3 * E_PAD + c0, SB * K)],
                         vbuf.at[p], isem)

    def drain_super():
        pltpu.make_async_copy(idata.at[pl.ds(0, SB * K)], idxc.at[0],
                              isem).wait()
        for j in range(SB):
            pltpu.make_async_copy(idata.at[pl.ds(0, K)], idxr.at[0, j],
                                  isem).wait()
        pltpu.make_async_copy(idata.at[pl.ds(0, SB * K)], vbuf.at[0],
                              isem).wait()

    def spmm(src, dst):
        fetch_super(0, 0)
        zd = [pltpu.async_copy(zbuf, acc.at[pl.ds(row0 + j * ZR, ZR)], zsem)
              for j in range(NB)]
        for d in zd:
            d.wait()
        plsc.subcore_barrier()

        def super_chunk(s, carry):
            p = s & 1
            drain_super()

            @pl.when(s + 1 < NSUP)
            def _():
                fetch_super(s + 1, 1 - p)

            gd = [None, None]
            sd = [None, None]
            gd[0] = pltpu.async_copy(src.at[idxc.at[p, pl.ds(0, K)]],
                                     gbufs[0], gsems[0])
            for j in range(SB):
                b = j & 1
                if j + 1 < SB:
                    gd[1 - b] = pltpu.async_copy(
                        src.at[idxc.at[p, pl.ds((j + 1) * K, K)]],
                        gbufs[1 - b], gsems[1 - b])
                gd[b].wait()
                if j >= 2:
                    sd[b].wait()

                def scale_group(g, c2, _j=j, _b=b):
                    vv = plsc.bitcast(vbuf[p, pl.ds(_j * K + g * 16, 16)],
                                      jnp.float32)
                    gb = gbufs[_b]
                    sb = sbufs[_b]
                    for e in range(16):
                        i = g * 16 + e
                        v = vv[e]
                        for h in range(0, H, 16):
                            sl = pl.ds(h, 16)
                            sb[i, sl] = gb[i, sl] * v
                    return c2

                lax.fori_loop(0, K // 16, scale_group, 0)
                sd[b] = pltpu.async_copy(sbufs[b], acc.at[idxr.at[p, j]],
                                         ssems[b], add=True)
            sd[0].wait()
            sd[1].wait()
            return carry

        lax.fori_loop(0, NSUP, super_chunk, 0)
        plsc.subcore_barrier()

        wd = [pltpu.async_copy(acc.at[pl.ds(row0 + j * ZR, ZR)],
                               dst.at[pl.ds(base_col + row0 + j * ZR, ZR)],
                               zsem)
              for j in range(NB)]
        for d in wd:
            d.wait()
        plsc.subcore_barrier()

    spmm(x0, y1)
    spmm(y1, y2)
    spmm(y2, y3)

    def mean_rows(nrows, goff):
        sl_rows = pl.ds(goff, nrows)
        pltpu.sync_copy(x0.at[sl_rows], gbuf0.at[pl.ds(0, nrows)])
        pltpu.sync_copy(y1.at[sl_rows], gbuf1.at[pl.ds(0, nrows)])
        pltpu.sync_copy(y2.at[sl_rows], sbuf0.at[pl.ds(0, nrows)])
        pltpu.sync_copy(y3.at[sl_rows], sbuf1.at[pl.ds(0, nrows)])

        def mean_row(i, c2):
            for h in range(0, H, 16):
                sl = pl.ds(h, 16)
                gbuf0[i, sl] = (gbuf0[i, sl] + gbuf1[i, sl] + sbuf0[i, sl]
                                + sbuf1[i, sl]) * 0.25
            return c2

        lax.fori_loop(0, nrows, mean_row, 0)
        pltpu.sync_copy(gbuf0.at[pl.ds(0, nrows)], mn.at[sl_rows])

    MR = K
    NMB = NR // MR
    MT = NR - NMB * MR

    def mean_block(j, carry):
        mean_rows(MR, base_col + row0 + j * MR)
        return carry

    lax.fori_loop(0, NMB, mean_block, 0)
    mean_rows(MT, base_col + row0 + NMB * MR)
    plsc.subcore_barrier()

    spmm(mn, mie)


def kernel(user_emb, item_emb, adj_indices, adj_values):
    ego = jnp.concatenate([user_emb, item_emb], axis=0)
    rows = adj_indices[0].astype(jnp.int32)
    cols = adj_indices[1].astype(jnp.int32)
    vals = adj_values.astype(jnp.float32)

    pad = E_PAD - E
    pidx = (jnp.arange(pad, dtype=jnp.int32) * 61) % N
    rows_p = jnp.concatenate([rows, pidx])
    cols_p = jnp.concatenate([cols, pidx])
    vals_p = jnp.concatenate([vals, jnp.zeros((pad,), jnp.float32)])

    zpad = jnp.zeros((NP - N, H), jnp.float32)
    x0 = jnp.concatenate([ego[:, :H], zpad, ego[:, H:], zpad], axis=0)

    mesh = plsc.VectorSubcoreMesh(core_axis_name="c", subcore_axis_name="s")
    out_type = jax.ShapeDtypeStruct((5, 2 * NP, H), jnp.float32)
    scratch = [
        pltpu.VMEM_SHARED((NP, H), jnp.float32),
        pltpu.VMEM((2, SB * K), jnp.int32),
        pltpu.VMEM((2, SB, K), jnp.int32),
        pltpu.VMEM((2, SB * K), jnp.int32),
        pltpu.VMEM((K, H), jnp.float32),
        pltpu.VMEM((K, H), jnp.float32),
        pltpu.VMEM((K, H), jnp.float32),
        pltpu.VMEM((K, H), jnp.float32),
        pltpu.VMEM((ZR, H), jnp.float32),
        pltpu.SemaphoreType.DMA,
        pltpu.SemaphoreType.DMA,
        pltpu.SemaphoreType.DMA,
        pltpu.SemaphoreType.DMA,
        pltpu.SemaphoreType.DMA,
        pltpu.SemaphoreType.DMA,
    ]
    run = pl.kernel(_sc_body, out_type=out_type, mesh=mesh,
                    scratch_types=scratch,
                    compiler_params=pltpu.CompilerParams(
                        use_tc_tiling_on_sc=False,
                        needs_layout_passes=False))
    idata = jnp.concatenate([
        cols_p, cols_p + NP, rows_p,
        jax.lax.bitcast_convert_type(vals_p, jnp.int32)])
    Y = run(x0, idata)

    def unsplit(t):
        return jnp.concatenate([t[:N], t[NP:NP + N]], axis=1)

    l1, l2, l3 = unsplit(Y[0]), unsplit(Y[1]), unsplit(Y[2])
    mean_emb = unsplit(Y[3])
    mean_item_emb = unsplit(Y[4])
    stacked = jnp.stack([ego, l1, l2, l3], axis=1)
    return (mean_emb[:N_USERS], mean_emb[N_USERS:], stacked, mean_item_emb)
